# baseline (speedup 1.0000x reference)
import functools

import jax
import jax.numpy as jnp
from jax.experimental import pallas as pl
from jax.experimental.pallas import tpu as pltpu

KNN = 50
M = 100000
MB = 2048
NBLK = (M + MB - 1) // MB
MPAD = NBLK * MB
BT = 128
NBT = 1024 // BT
SUB = 128
NSLOT = 8
NSUB = NSLOT * SUB
KAUG = 72
DELTA = 1e-3
NEG_INF = float("-inf")
SENTINEL = 3e38


def _head_kernel(x_ref, w1_ref, b1_ref, wp_ref, bp_ref, h_ref, probs_ref, haug_ref):
    x = x_ref[...]
    h = jnp.maximum(
        jax.lax.dot_general(x, w1_ref[...], (((1,), (1,)), ((), ()))) + b1_ref[...][None, :],
        0.0,
    )
    h_ref[...] = h
    logits = jax.lax.dot_general(h, wp_ref[...], (((1,), (1,)), ((), ()))) + bp_ref[...][None, :]
    m = jnp.max(logits, axis=1, keepdims=True)
    e = jnp.exp(logits - m)
    probs_ref[...] = e / jnp.sum(e, axis=1, keepdims=True)
    hh = jnp.sum(h * h, axis=1, keepdims=True)
    lane = jax.lax.broadcasted_iota(jnp.int32, (x.shape[0], KAUG), 1)
    H = h.shape[1]
    haug = jnp.where(lane < H, jnp.pad(-2.0 * h, ((0, 0), (0, KAUG - H))), 0.0)
    haug = jnp.where(lane == H, hh, haug)
    haug = jnp.where(lane == H + 1, 1.0, haug)
    haug_ref[...] = haug


def _kk_kernel(kt_ref, kk_ref):
    j = pl.program_id(0)
    kt = kt_ref[...]
    kk = jnp.sum(kt * kt, axis=0, keepdims=True)
    col = j * MB + jax.lax.broadcasted_iota(jnp.int32, kk.shape, 1)
    kk_ref[...] = jnp.where(col < M, kk, SENTINEL)


def _dnd_kernel(haug_ref, kaug_ref, vals_ref, out_ref, slab_ref, bm_ref):
    j = pl.program_id(1)

    @pl.when(j < NBLK)
    def _gen():
        d2 = jax.lax.dot_general(haug_ref[...], kaug_ref[...], (((1,), (0,)), ((), ())))
        slab_ref[:, pl.ds(j * MB, MB)] = d2
        bm = jnp.maximum(jnp.min(d2.reshape(BT, MB // SUB, SUB), axis=1), 0.0)
        off = (j % NSLOT) * SUB
        old = jnp.where(j < NSLOT, jnp.float32(jnp.inf), bm_ref[:, pl.ds(off, SUB)])
        bm_ref[:, pl.ds(off, SUB)] = jnp.minimum(bm, old)

    @pl.when(j == NBLK)
    def _select():
        bmbits = jax.lax.bitcast_convert_type(bm_ref[...], jnp.int32)

        def bs_body(_, lohi):
            lo, hi = lohi
            mid = lo + jax.lax.shift_right_arithmetic(hi - lo, 1)
            cnt = jnp.sum((bmbits <= mid).astype(jnp.float32), axis=1, keepdims=True)
            ge = cnt >= KNN
            return jnp.where(ge, lo, mid), jnp.where(ge, mid, hi)

        lo0 = jnp.full((BT, 1), -1, jnp.int32)
        hi0 = jnp.full((BT, 1), 0x7F800000, jnp.int32)
        _, hi = jax.lax.fori_loop(0, 31, bs_body, (lo0, hi0))
        t_ub = jax.lax.bitcast_convert_type(hi, jnp.float32)

        def sum_body(c, carry):
            swv_, swvv_, nv_, m1a, c1a, v1a, m2a, c2a, v2a = carry
            d2 = slab_ref[:, pl.ds(c * MB, MB)]
            v = vals_ref[0, pl.ds(c * MB, MB)][None, :]
            mask = d2 <= t_ub
            r = 1.0 / (d2 + DELTA)
            wpart = jnp.where(mask, r, 0.0)
            wvpart = jnp.where(mask, r * v, 0.0)
            npart = mask.astype(jnp.float32)
            swv_ = swv_ + jnp.sum(wpart.reshape(BT, MB // SUB, SUB), axis=1)
            swvv_ = swvv_ + jnp.sum(wvpart.reshape(BT, MB // SUB, SUB), axis=1)
            nv_ = nv_ + jnp.sum(npart.reshape(BT, MB // SUB, SUB), axis=1)
            xm = jnp.where(mask, d2, NEG_INF)
            m1 = jnp.max(xm, axis=1, keepdims=True)
            ok1 = m1 > NEG_INF
            eq1 = xm == m1
            c1 = jnp.where(ok1, jnp.sum(eq1.astype(jnp.float32), axis=1, keepdims=True), 0.0)
            v1 = jnp.where(ok1, jnp.sum(jnp.where(eq1, v, 0.0), axis=1, keepdims=True), 0.0)
            xm2 = jnp.where(eq1, NEG_INF, xm)
            m2 = jnp.max(xm2, axis=1, keepdims=True)
            ok2 = m2 > NEG_INF
            eq2 = xm2 == m2
            c2 = jnp.where(ok2, jnp.sum(eq2.astype(jnp.float32), axis=1, keepdims=True), 0.0)
            v2 = jnp.where(ok2, jnp.sum(jnp.where(eq2, v, 0.0), axis=1, keepdims=True), 0.0)
            lm = jax.lax.broadcasted_iota(jnp.int32, (BT, 64), 1) == c
            m1a = jnp.where(lm, m1, m1a)
            c1a = jnp.where(lm, c1, c1a)
            v1a = jnp.where(lm, v1, v1a)
            m2a = jnp.where(lm, m2, m2a)
            c2a = jnp.where(lm, c2, c2a)
            v2a = jnp.where(lm, v2, v2a)
            return swv_, swvv_, nv_, m1a, c1a, v1a, m2a, c2a, v2a

        z = jnp.zeros((BT, 1), jnp.float32)
        zl = jnp.zeros((BT, SUB), jnp.float32)
        z64 = jnp.zeros((BT, 64), jnp.float32)
        ninf64 = jnp.full((BT, 64), NEG_INF, jnp.float32)
        swv_, swvv_, nv_, m1a, c1a, v1a, m2a, c2a, v2a = jax.lax.fori_loop(
            0, NBLK, sum_body, (zl, zl, zl, ninf64, z64, z64, ninf64, z64, z64))
        s_w = jnp.sum(swv_, axis=1, keepdims=True)
        s_wv = jnp.sum(swvv_, axis=1, keepdims=True)
        n = jnp.sum(nv_, axis=1, keepdims=True)
        excess = n - KNN
        cap = t_ub

        def rpeel_body(_, state):
            k, excess, s_w, s_wv, cap = state
            avail = jnp.where(k == 0.0, m1a, jnp.where(k == 1.0, m2a, NEG_INF))
            ub = jnp.where(k >= 2.0, m2a, NEG_INF)
            g = jnp.max(avail, axis=1, keepdims=True)
            u = jnp.max(ub, axis=1, keepdims=True)
            act = (excess > 0.0) & (g > NEG_INF) & (u <= g)
            match = (avail == g) & act
            cnt_at = jnp.where(k == 0.0, c1a, c2a)
            sv_at = jnp.where(k == 0.0, v1a, v2a)
            totcnt = jnp.sum(jnp.where(match, cnt_at, 0.0), axis=1, keepdims=True)
            totsv = jnp.sum(jnp.where(match, sv_at, 0.0), axis=1, keepdims=True)
            w_g = 1.0 / (g + DELTA)
            take = jnp.minimum(totcnt, excess)
            s_w = jnp.where(act, s_w - w_g * take, s_w)
            s_wv = jnp.where(act, s_wv - w_g * totsv * (take / totcnt), s_wv)
            excess = jnp.where(act, excess - take, excess)
            k = k + match.astype(jnp.float32)
            gbits = jax.lax.bitcast_convert_type(g, jnp.int32)
            g_dec = jax.lax.bitcast_convert_type(gbits - 1, jnp.float32)
            cap = jnp.where(act & (g > 0.0), g_dec, jnp.where(act, -1.0, cap))
            return k, excess, s_w, s_wv, cap

        _, excess, s_w, s_wv, cap = jax.lax.fori_loop(
            0, 16, rpeel_body, (z64, excess, s_w, s_wv, cap))

        def peel_cond(state):
            _, excess, _, _ = state
            return jnp.any(excess > 0.0)

        def peel_body(state):
            cap, excess, s_w, s_wv = state

            def max_body(c, carry):
                m, cnt, sv = carry
                d2 = slab_ref[:, pl.ds(c * MB, MB)]
                v = vals_ref[0, pl.ds(c * MB, MB)][None, :]
                xm = jnp.where(d2 <= cap, d2, NEG_INF)
                cm = jnp.max(xm, axis=1, keepdims=True)
                eq = xm == cm
                cntc = jnp.sum(eq.astype(jnp.float32), axis=1, keepdims=True)
                svc = jnp.sum(jnp.where(eq, v, 0.0), axis=1, keepdims=True)
                valid = cm > NEG_INF
                gt = valid & (cm > m)
                same = valid & (cm == m)
                m2 = jnp.where(gt, cm, m)
                cnt2 = jnp.where(gt, cntc, jnp.where(same, cnt + cntc, cnt))
                sv2 = jnp.where(gt, svc, jnp.where(same, sv + svc, sv))
                return m2, cnt2, sv2

            m0 = jnp.full((BT, 1), NEG_INF, jnp.float32)
            m, cnt, sv = jax.lax.fori_loop(0, NBLK, max_body, (m0, z, z))

            act = excess > 0.0
            w_m = 1.0 / (m + DELTA)
            take = jnp.minimum(cnt, excess)
            s_w2 = jnp.where(act, s_w - w_m * take, s_w)
            s_wv2 = jnp.where(act, s_wv - w_m * sv * (take / cnt), s_wv)
            excess2 = jnp.where(act, excess - take, excess)
            mbits = jax.lax.bitcast_convert_type(m, jnp.int32)
            cap_dec = jax.lax.bitcast_convert_type(mbits - 1, jnp.float32)
            cap2 = jnp.where(act & (m > 0.0), cap_dec, jnp.where(act, -1.0, cap))
            return cap2, excess2, s_w2, s_wv2

        _, _, s_w, s_wv = jax.lax.while_loop(peel_cond, peel_body, (cap, excess, s_w, s_wv))
        out_ref[...] = s_wv / s_w


def kernel(x, W1, b1, Wp, bp, dnd_keys, dnd_vals):
    B = x.shape[0]
    H = W1.shape[0]
    A = Wp.shape[0]

    h, probs, haug = pl.pallas_call(
        _head_kernel,
        out_shape=(
            jax.ShapeDtypeStruct((B, H), jnp.float32),
            jax.ShapeDtypeStruct((B, A), jnp.float32),
            jax.ShapeDtypeStruct((B, KAUG), jnp.float32),
        ),
    )(x, W1, b1, Wp, bp)

    keys_t = jnp.pad(dnd_keys, ((0, MPAD - M), (0, 0))).T
    kk = pl.pallas_call(
        _kk_kernel,
        grid=(NBLK,),
        in_specs=[pl.BlockSpec((H, MB), lambda j: (0, j))],
        out_specs=pl.BlockSpec((1, MB), lambda j: (0, j)),
        out_shape=jax.ShapeDtypeStruct((1, MPAD), jnp.float32),
    )(keys_t)
    kaug = jnp.concatenate(
        [keys_t, jnp.ones((1, MPAD), jnp.float32), kk,
         jnp.zeros((KAUG - H - 2, MPAD), jnp.float32)], axis=0)
    vals_pad = jnp.pad(dnd_vals[:, 0], (0, MPAD - M)).reshape(1, MPAD)

    value = pl.pallas_call(
        _dnd_kernel,
        grid=(NBT, NBLK + 1),
        in_specs=[
            pl.BlockSpec((BT, KAUG), lambda i, j: (i, 0)),
            pl.BlockSpec((KAUG, MB), lambda i, j: (0, jnp.minimum(j, NBLK - 1))),
            pl.BlockSpec((1, MPAD), lambda i, j: (0, 0)),
        ],
        out_specs=pl.BlockSpec((BT, 1), lambda i, j: (i, 0)),
        out_shape=jax.ShapeDtypeStruct((B, 1), jnp.float32),
        scratch_shapes=[
            pltpu.VMEM((BT, MPAD), jnp.float32),
            pltpu.VMEM((BT, NSUB), jnp.float32),
        ],
    )(haug, kaug, vals_pad)

    return probs, value, h


# speedup vs baseline: 1.4273x; 1.4273x over previous
import functools

import jax
import jax.numpy as jnp
from jax.experimental import pallas as pl
from jax.experimental.pallas import tpu as pltpu

KNN = 50
M = 100000
MB = 2048
NBLK = (M + MB - 1) // MB
MPAD = NBLK * MB
BT = 128
NBT = 1024 // BT
SUB = 128
NSLOT = 8
NSUB = NSLOT * SUB
DELTA = 1e-3
NEG_INF = float("-inf")
SENTINEL = 3e38


def _head_kernel(x_ref, w1_ref, b1_ref, wp_ref, bp_ref, h_ref, probs_ref, hh_ref):
    x = x_ref[...]
    h = jnp.maximum(
        jax.lax.dot_general(x, w1_ref[...], (((1,), (1,)), ((), ()))) + b1_ref[...][None, :],
        0.0,
    )
    h_ref[...] = h
    logits = jax.lax.dot_general(h, wp_ref[...], (((1,), (1,)), ((), ()))) + bp_ref[...][None, :]
    m = jnp.max(logits, axis=1, keepdims=True)
    e = jnp.exp(logits - m)
    probs_ref[...] = e / jnp.sum(e, axis=1, keepdims=True)
    hh_ref[...] = jnp.sum(h * h, axis=1, keepdims=True)


def _kk_kernel(keys_ref, kk_ref):
    j = pl.program_id(0)
    keys = keys_ref[...]
    kk = jnp.sum(keys * keys, axis=1)[None, :]
    col = j * MB + jax.lax.broadcasted_iota(jnp.int32, kk.shape, 1)
    kk_ref[...] = jnp.where(col < M, kk, SENTINEL)


def _dnd_kernel(h_ref, hh_ref, keys_ref, kk_ref, vals_ref, out_ref, slab_ref, bm_ref):
    j = pl.program_id(1)

    @pl.when(j < NBLK)
    def _gen():
        h = h_ref[...]
        keys = keys_ref[...]
        cross = jax.lax.dot_general(h, keys, (((1,), (1,)), ((), ())))
        d2 = hh_ref[...] + kk_ref[...] - 2.0 * cross
        slab_ref[:, pl.ds(j * MB, MB)] = d2
        bm = jnp.maximum(jnp.min(d2.reshape(BT, MB // SUB, SUB), axis=1), 0.0)
        off = (j % NSLOT) * SUB
        old = jnp.where(j < NSLOT, jnp.float32(jnp.inf), bm_ref[:, pl.ds(off, SUB)])
        bm_ref[:, pl.ds(off, SUB)] = jnp.minimum(bm, old)

    @pl.when(j == NBLK)
    def _select():
        bmbits = jax.lax.bitcast_convert_type(bm_ref[...], jnp.int32)

        def bs_body(_, lohi):
            lo, hi = lohi
            mid = lo + jax.lax.shift_right_arithmetic(hi - lo, 1)
            cnt = jnp.sum((bmbits <= mid).astype(jnp.float32), axis=1, keepdims=True)
            ge = cnt >= KNN
            return jnp.where(ge, lo, mid), jnp.where(ge, mid, hi)

        lo0 = jnp.full((BT, 1), -1, jnp.int32)
        hi0 = jnp.full((BT, 1), 0x7F800000, jnp.int32)
        _, hi = jax.lax.fori_loop(0, 31, bs_body, (lo0, hi0))
        t_ub = jax.lax.bitcast_convert_type(hi, jnp.float32)

        def sum_body(c, carry):
            s_w, s_wv, n, m1a, c1a, v1a, m2a, c2a, v2a = carry
            d2 = slab_ref[:, pl.ds(c * MB, MB)]
            v = vals_ref[0, pl.ds(c * MB, MB)][None, :]
            mask = d2 <= t_ub
            r = 1.0 / (d2 + DELTA)
            s_w = s_w + jnp.sum(jnp.where(mask, r, 0.0), axis=1, keepdims=True)
            s_wv = s_wv + jnp.sum(jnp.where(mask, r * v, 0.0), axis=1, keepdims=True)
            n = n + jnp.sum(mask.astype(jnp.float32), axis=1, keepdims=True)
            xm = jnp.where(mask, d2, NEG_INF)
            m1 = jnp.max(xm, axis=1, keepdims=True)
            ok1 = m1 > NEG_INF
            eq1 = xm == m1
            c1 = jnp.where(ok1, jnp.sum(eq1.astype(jnp.float32), axis=1, keepdims=True), 0.0)
            v1 = jnp.where(ok1, jnp.sum(jnp.where(eq1, v, 0.0), axis=1, keepdims=True), 0.0)
            xm2 = jnp.where(eq1, NEG_INF, xm)
            m2 = jnp.max(xm2, axis=1, keepdims=True)
            ok2 = m2 > NEG_INF
            eq2 = xm2 == m2
            c2 = jnp.where(ok2, jnp.sum(eq2.astype(jnp.float32), axis=1, keepdims=True), 0.0)
            v2 = jnp.where(ok2, jnp.sum(jnp.where(eq2, v, 0.0), axis=1, keepdims=True), 0.0)
            lm = jax.lax.broadcasted_iota(jnp.int32, (BT, 64), 1) == c
            m1a = jnp.where(lm, m1, m1a)
            c1a = jnp.where(lm, c1, c1a)
            v1a = jnp.where(lm, v1, v1a)
            m2a = jnp.where(lm, m2, m2a)
            c2a = jnp.where(lm, c2, c2a)
            v2a = jnp.where(lm, v2, v2a)
            return s_w, s_wv, n, m1a, c1a, v1a, m2a, c2a, v2a

        z = jnp.zeros((BT, 1), jnp.float32)
        z64 = jnp.zeros((BT, 64), jnp.float32)
        ninf64 = jnp.full((BT, 64), NEG_INF, jnp.float32)
        s_w, s_wv, n, m1a, c1a, v1a, m2a, c2a, v2a = jax.lax.fori_loop(
            0, NBLK, sum_body, (z, z, z, ninf64, z64, z64, ninf64, z64, z64))
        excess = n - KNN
        cap = t_ub

        def rpeel_body(_, state):
            k, excess, s_w, s_wv, cap = state
            avail = jnp.where(k == 0.0, m1a, jnp.where(k == 1.0, m2a, NEG_INF))
            ub = jnp.where(k >= 2.0, m2a, NEG_INF)
            g = jnp.max(avail, axis=1, keepdims=True)
            u = jnp.max(ub, axis=1, keepdims=True)
            act = (excess > 0.0) & (g > NEG_INF) & (u <= g)
            match = (avail == g) & act
            cnt_at = jnp.where(k == 0.0, c1a, c2a)
            sv_at = jnp.where(k == 0.0, v1a, v2a)
            totcnt = jnp.sum(jnp.where(match, cnt_at, 0.0), axis=1, keepdims=True)
            totsv = jnp.sum(jnp.where(match, sv_at, 0.0), axis=1, keepdims=True)
            w_g = 1.0 / (g + DELTA)
            take = jnp.minimum(totcnt, excess)
            s_w = jnp.where(act, s_w - w_g * take, s_w)
            s_wv = jnp.where(act, s_wv - w_g * totsv * (take / totcnt), s_wv)
            excess = jnp.where(act, excess - take, excess)
            k = k + match.astype(jnp.float32)
            gbits = jax.lax.bitcast_convert_type(g, jnp.int32)
            g_dec = jax.lax.bitcast_convert_type(gbits - 1, jnp.float32)
            cap = jnp.where(act & (g > 0.0), g_dec, jnp.where(act, -1.0, cap))
            return k, excess, s_w, s_wv, cap

        _, excess, s_w, s_wv, cap = jax.lax.fori_loop(
            0, 16, rpeel_body, (z64, excess, s_w, s_wv, cap))

        def peel_cond(state):
            _, excess, _, _ = state
            return jnp.any(excess > 0.0)

        def peel_body(state):
            cap, excess, s_w, s_wv = state

            def max_body(c, carry):
                m, cnt, sv = carry
                d2 = slab_ref[:, pl.ds(c * MB, MB)]
                v = vals_ref[0, pl.ds(c * MB, MB)][None, :]
                xm = jnp.where(d2 <= cap, d2, NEG_INF)
                cm = jnp.max(xm, axis=1, keepdims=True)
                eq = xm == cm
                cntc = jnp.sum(eq.astype(jnp.float32), axis=1, keepdims=True)
                svc = jnp.sum(jnp.where(eq, v, 0.0), axis=1, keepdims=True)
                valid = cm > NEG_INF
                gt = valid & (cm > m)
                same = valid & (cm == m)
                m2 = jnp.where(gt, cm, m)
                cnt2 = jnp.where(gt, cntc, jnp.where(same, cnt + cntc, cnt))
                sv2 = jnp.where(gt, svc, jnp.where(same, sv + svc, sv))
                return m2, cnt2, sv2

            m0 = jnp.full((BT, 1), NEG_INF, jnp.float32)
            m, cnt, sv = jax.lax.fori_loop(0, NBLK, max_body, (m0, z, z))

            act = excess > 0.0
            w_m = 1.0 / (m + DELTA)
            take = jnp.minimum(cnt, excess)
            s_w2 = jnp.where(act, s_w - w_m * take, s_w)
            s_wv2 = jnp.where(act, s_wv - w_m * sv * (take / cnt), s_wv)
            excess2 = jnp.where(act, excess - take, excess)
            mbits = jax.lax.bitcast_convert_type(m, jnp.int32)
            cap_dec = jax.lax.bitcast_convert_type(mbits - 1, jnp.float32)
            cap2 = jnp.where(act & (m > 0.0), cap_dec, jnp.where(act, -1.0, cap))
            return cap2, excess2, s_w2, s_wv2

        _, _, s_w, s_wv = jax.lax.while_loop(peel_cond, peel_body, (cap, excess, s_w, s_wv))
        out_ref[...] = s_wv / s_w


def kernel(x, W1, b1, Wp, bp, dnd_keys, dnd_vals):
    B = x.shape[0]
    H = W1.shape[0]
    A = Wp.shape[0]

    h, probs, hh = pl.pallas_call(
        _head_kernel,
        out_shape=(
            jax.ShapeDtypeStruct((B, H), jnp.float32),
            jax.ShapeDtypeStruct((B, A), jnp.float32),
            jax.ShapeDtypeStruct((B, 1), jnp.float32),
        ),
    )(x, W1, b1, Wp, bp)

    keys_pad = jnp.pad(dnd_keys, ((0, MPAD - M), (0, 0)))
    vals_pad = jnp.pad(dnd_vals[:, 0], (0, MPAD - M)).reshape(1, MPAD)
    kk = pl.pallas_call(
        _kk_kernel,
        grid=(NBLK,),
        in_specs=[pl.BlockSpec((MB, H), lambda j: (j, 0))],
        out_specs=pl.BlockSpec((1, MB), lambda j: (0, j)),
        out_shape=jax.ShapeDtypeStruct((1, MPAD), jnp.float32),
    )(keys_pad)

    value = pl.pallas_call(
        _dnd_kernel,
        grid=(NBT, NBLK + 1),
        in_specs=[
            pl.BlockSpec((BT, H), lambda i, j: (i, 0)),
            pl.BlockSpec((BT, 1), lambda i, j: (i, 0)),
            pl.BlockSpec((MB, H), lambda i, j: (jnp.minimum(j, NBLK - 1), 0)),
            pl.BlockSpec((1, MB), lambda i, j: (0, jnp.minimum(j, NBLK - 1))),
            pl.BlockSpec((1, MPAD), lambda i, j: (0, 0)),
        ],
        out_specs=pl.BlockSpec((BT, 1), lambda i, j: (i, 0)),
        out_shape=jax.ShapeDtypeStruct((B, 1), jnp.float32),
        scratch_shapes=[
            pltpu.VMEM((BT, MPAD), jnp.float32),
            pltpu.VMEM((BT, NSUB), jnp.float32),
        ],
    )(h, hh, keys_pad, kk, vals_pad)

    return probs, value, h
